# Initial kernel scaffold; baseline (speedup 1.0000x reference)
import jax
import jax.numpy as jnp
from jax.experimental import pallas as pl


def kernel(h, edge_index, W1, b1, W2, b2, W3, b3, W4, b4, W5, b5, W6, b6, W7, b7, W8, b8, Wc, bc):
    raise NotImplementedError("write your pallas kernel here")


# speedup vs baseline: 2.5453x; 2.5453x over previous
import functools

import jax
import jax.numpy as jnp
from jax import lax
from jax.experimental import pallas as pl
from jax.experimental.pallas import tpu as pltpu
from jax.experimental.pallas import tpu_sc as plsc

N = 10000
NPAD = 10240
E = 160000
HID = 512
NCHUNK = 4
CW = HID // NCHUNK
NS = 16
NC = 2
EP = E // NS
K = 80
NB = EP // K
RPT = NPAD // NS
ZR = 128


def _sc_mesh():
    return plsc.VectorSubcoreMesh(core_axis_name="c", subcore_axis_name="s")


def _deg_call(edges_flat):
    @functools.partial(
        pl.kernel,
        out_type=jax.ShapeDtypeStruct((NC, NPAD, CW), jnp.float32),
        mesh=_sc_mesh(),
        scratch_types=[
            pltpu.VMEM((K,), jnp.int32),
            pltpu.VMEM((K, CW), jnp.float32),
            pltpu.VMEM((ZR, CW), jnp.float32),
            pltpu.VMEM_SHARED((NPAD, CW), jnp.float32),
        ],
    )
    def deg_kernel(edges_hbm, deg_hbm, idx_v, ones_v, zbuf_v, acc_sh):
        cid = lax.axis_index("c")
        sid = lax.axis_index("s")

        def fill_ones(r, c):
            for j in range(CW // 16):
                ones_v[r, pl.ds(j * 16, 16)] = jnp.ones((16,), jnp.float32)
            return c

        lax.fori_loop(0, K, fill_ones, 0)

        def fill_zero(r, c):
            for j in range(CW // 16):
                zbuf_v[r, pl.ds(j * 16, 16)] = jnp.zeros((16,), jnp.float32)
            return c

        lax.fori_loop(0, ZR, fill_zero, 0)

        for z in range(RPT // ZR):
            pltpu.sync_copy(zbuf_v, acc_sh.at[pl.ds(sid * RPT + z * ZR, ZR)])
        plsc.subcore_barrier()

        def body(b, c):
            base = cid * E + sid * EP + b * K
            pltpu.sync_copy(edges_hbm.at[pl.ds(base, K)], idx_v)
            pltpu.sync_copy(ones_v, acc_sh.at[idx_v], add=True)
            return c

        lax.fori_loop(0, NB, body, 0)
        plsc.subcore_barrier()
        pltpu.sync_copy(acc_sh.at[pl.ds(sid * RPT, RPT)],
                        deg_hbm.at[cid, pl.ds(sid * RPT, RPT)])

    return deg_kernel(edges_flat)


def _mp_call(y4, srcidx_flat, dst):
    @functools.partial(
        pl.kernel,
        out_type=jax.ShapeDtypeStruct((NCHUNK, NPAD, CW), jnp.float32),
        mesh=_sc_mesh(),
        scratch_types=[
            pltpu.VMEM((K,), jnp.int32),
            pltpu.VMEM((K,), jnp.int32),
            pltpu.VMEM((K, CW), jnp.float32),
            pltpu.VMEM((ZR, CW), jnp.float32),
            pltpu.VMEM_SHARED((NPAD, CW), jnp.float32),
            pltpu.SemaphoreType.DMA,
        ],
    )
    def mp_kernel(y_hbm, si_hbm, dst_hbm, out_hbm,
                  sidx_v, didx_v, rows_v, zbuf_v, acc_sh, gsem):
        cid = lax.axis_index("c")
        sid = lax.axis_index("s")

        def fill_zero(r, c):
            for j in range(CW // 16):
                zbuf_v[r, pl.ds(j * 16, 16)] = jnp.zeros((16,), jnp.float32)
            return c

        lax.fori_loop(0, ZR, fill_zero, 0)

        for cc in range(NCHUNK // NC):
            chunk = cid * (NCHUNK // NC) + cc
            for z in range(RPT // ZR):
                pltpu.sync_copy(zbuf_v,
                                acc_sh.at[pl.ds(sid * RPT + z * ZR, ZR)])
            plsc.subcore_barrier()

            def body(b, c):
                base = sid * EP + b * K
                pltpu.sync_copy(si_hbm.at[pl.ds(chunk * E + base, K)], sidx_v)
                pltpu.sync_copy(dst_hbm.at[pl.ds(base, K)], didx_v)
                pltpu.async_copy(y_hbm.at[sidx_v], rows_v, gsem).wait()
                pltpu.sync_copy(rows_v, acc_sh.at[didx_v], add=True)
                return c

            lax.fori_loop(0, NB, body, 0)
            plsc.subcore_barrier()
            pltpu.sync_copy(acc_sh.at[pl.ds(sid * RPT, RPT)],
                            out_hbm.at[chunk, pl.ds(sid * RPT, RPT)])
            plsc.subcore_barrier()

    return mp_kernel(y4, srcidx_flat, dst)


BN = 1000


def _norm(deg_blk):
    return lax.rsqrt(jnp.maximum(deg_blk[:, :1], 1.0))


def _layer1_body(h_ref, dout_ref, w_ref, y_ref):
    ns = _norm(dout_ref[...])
    y_ref[...] = jnp.dot(h_ref[...] * ns, w_ref[...],
                         preferred_element_type=jnp.float32)


def _layer1_call(h, deg_out16, W1):
    ind = h.shape[1]
    return pl.pallas_call(
        _layer1_body,
        grid=(N // BN,),
        in_specs=[
            pl.BlockSpec((BN, ind), lambda i: (i, 0)),
            pl.BlockSpec((BN, CW), lambda i: (i, 0)),
            pl.BlockSpec((ind, HID), lambda i: (0, 0)),
        ],
        out_specs=pl.BlockSpec((BN, HID), lambda i: (i, 0)),
        out_shape=jax.ShapeDtypeStruct((N, HID), jnp.float32),
    )(h, deg_out16, W1)


def _fused_body(agg_ref, din_ref, dout_ref, b_ref, w_ref, y_ref):
    nd = _norm(din_ref[...])
    ns = _norm(dout_ref[...])
    acc = None
    for c in range(NCHUNK):
        xc = jnp.maximum(agg_ref[c] * nd + b_ref[:, c * CW:(c + 1) * CW],
                         0.0) * ns
        p = jnp.dot(xc, w_ref[c * CW:(c + 1) * CW, :],
                    preferred_element_type=jnp.float32)
        acc = p if acc is None else acc + p
    y_ref[...] = acc


def _fused_call(agg4, deg_in16, deg_out16, b2d, W):
    return pl.pallas_call(
        _fused_body,
        grid=(N // BN,),
        in_specs=[
            pl.BlockSpec((NCHUNK, BN, CW), lambda i: (0, i, 0)),
            pl.BlockSpec((BN, CW), lambda i: (i, 0)),
            pl.BlockSpec((BN, CW), lambda i: (i, 0)),
            pl.BlockSpec((1, HID), lambda i: (0, 0)),
            pl.BlockSpec((HID, HID), lambda i: (0, 0)),
        ],
        out_specs=pl.BlockSpec((BN, HID), lambda i: (i, 0)),
        out_shape=jax.ShapeDtypeStruct((N, HID), jnp.float32),
    )(agg4, deg_in16, deg_out16, b2d, W)


def _final_body(agg_ref, din_ref, b_ref, wc_ref, bc_ref, out_ref, hg_ref):
    i = pl.program_id(0)
    nd = _norm(din_ref[...])
    parts = []
    for c in range(NCHUNK):
        xc = jnp.maximum(agg_ref[c] * nd + b_ref[:, c * CW:(c + 1) * CW], 0.0)
        parts.append(jnp.sum(xc, axis=0, keepdims=True))
    part = jnp.concatenate(parts, axis=1)

    @pl.when(i == 0)
    def _():
        hg_ref[...] = part

    @pl.when(i > 0)
    def _():
        hg_ref[...] = hg_ref[...] + part

    @pl.when(i == pl.num_programs(0) - 1)
    def _():
        out_ref[...] = jnp.dot(hg_ref[...], wc_ref[...],
                               preferred_element_type=jnp.float32) + bc_ref[...]


def _final_call(agg4, deg_in16, b2d, Wc, bc2d):
    ncls = Wc.shape[1]
    return pl.pallas_call(
        _final_body,
        grid=(N // BN,),
        in_specs=[
            pl.BlockSpec((NCHUNK, BN, CW), lambda i: (0, i, 0)),
            pl.BlockSpec((BN, CW), lambda i: (i, 0)),
            pl.BlockSpec((1, HID), lambda i: (0, 0)),
            pl.BlockSpec((HID, ncls), lambda i: (0, 0)),
            pl.BlockSpec((1, ncls), lambda i: (0, 0)),
        ],
        out_specs=pl.BlockSpec((1, ncls), lambda i: (0, 0)),
        out_shape=jax.ShapeDtypeStruct((1, ncls), jnp.float32),
        scratch_shapes=[pltpu.VMEM((1, HID), jnp.float32)],
    )(agg4, deg_in16, b2d, Wc, bc2d)


def kernel(h, edge_index, W1, b1, W2, b2, W3, b3, W4, b4, W5, b5, W6, b6,
           W7, b7, W8, b8, Wc, bc):
    edges = edge_index.astype(jnp.int32)
    src = edges[0]
    dst = edges[1]
    srcidx = (src[None, :] * NCHUNK
              + jnp.arange(NCHUNK, dtype=jnp.int32)[:, None]).reshape(-1)

    deg16 = _deg_call(edges.reshape(-1))
    deg_out16 = deg16[0]
    deg_in16 = deg16[1]

    y = _layer1_call(h, deg_out16, W1)
    bs = [b1, b2, b3, b4, b5, b6, b7, b8]
    Ws = [None, W2, W3, W4, W5, W6, W7, W8]
    for i in range(1, 8):
        agg4 = _mp_call(y.reshape(NCHUNK * N, CW), srcidx, dst)
        y = _fused_call(agg4, deg_in16, deg_out16, bs[i - 1].reshape(1, HID),
                        Ws[i])
    agg4 = _mp_call(y.reshape(NCHUNK * N, CW), srcidx, dst)
    return _final_call(agg4, deg_in16, bs[7].reshape(1, HID), Wc,
                       bc.reshape(1, -1))


# speedup vs baseline: 2.7400x; 1.0765x over previous
import functools

import jax
import jax.numpy as jnp
from jax import lax
from jax.experimental import pallas as pl
from jax.experimental.pallas import tpu as pltpu
from jax.experimental.pallas import tpu_sc as plsc

N = 10000
NPAD = 10240
E = 160000
EPAD = 163840
HID = 512
NCHUNK = 4
CW = HID // NCHUNK
NS = 16
NC = 2
TEP = EPAD // NS
K = 128
NB = TEP // K
RPT = NPAD // NS
ZR = 128


def _sc_mesh():
    return plsc.VectorSubcoreMesh(core_axis_name="c", subcore_axis_name="s")


def _zero_fill(buf):
    def fill(r, c):
        for j in range(CW // 16):
            buf[r, pl.ds(j * 16, 16)] = jnp.zeros((16,), jnp.float32)
        return c

    lax.fori_loop(0, ZR, fill, 0)


def _deg_call(edges_pad):
    @functools.partial(
        pl.kernel,
        out_type=jax.ShapeDtypeStruct((NC, NPAD, CW), jnp.float32),
        mesh=_sc_mesh(),
        scratch_types=[
            pltpu.VMEM((NB, 1, K), jnp.int32),
            pltpu.VMEM((K, CW), jnp.float32),
            pltpu.VMEM((ZR, CW), jnp.float32),
            pltpu.VMEM_SHARED((NPAD, CW), jnp.float32),
        ],
    )
    def deg_kernel(edges_hbm, deg_hbm, idx_v, ones_v, zbuf_v, acc_sh):
        cid = lax.axis_index("c")
        sid = lax.axis_index("s")

        def fill_ones(r, c):
            for j in range(CW // 16):
                ones_v[r, pl.ds(j * 16, 16)] = jnp.ones((16,), jnp.float32)
            return c

        lax.fori_loop(0, K, fill_ones, 0)
        _zero_fill(zbuf_v)

        for z in range(RPT // ZR):
            pltpu.sync_copy(zbuf_v, acc_sh.at[pl.ds(sid * RPT + z * ZR, ZR)])
        pltpu.sync_copy(edges_hbm.at[pl.ds((cid * NS + sid) * NB, NB)], idx_v)
        plsc.subcore_barrier()

        def body(b, c):
            pltpu.sync_copy(ones_v, acc_sh.at[idx_v.at[b, 0]], add=True)
            return c

        lax.fori_loop(0, NB, body, 0)
        plsc.subcore_barrier()
        pltpu.sync_copy(acc_sh.at[pl.ds(sid * RPT, RPT)],
                        deg_hbm.at[cid, pl.ds(sid * RPT, RPT)])

    return deg_kernel(edges_pad)


def _mp_call(y4, srcidx, dstidx):
    @functools.partial(
        pl.kernel,
        out_type=jax.ShapeDtypeStruct((NCHUNK, NPAD, CW), jnp.float32),
        mesh=_sc_mesh(),
        scratch_types=[
            pltpu.VMEM((NB, 1, K), jnp.int32),
            pltpu.VMEM((4, 1, K), jnp.int32),
            pltpu.VMEM((K, CW), jnp.float32),
            pltpu.VMEM((K, CW), jnp.float32),
            pltpu.VMEM_SHARED((NPAD, CW), jnp.float32),
            pltpu.SemaphoreType.DMA,
            pltpu.SemaphoreType.DMA,
            pltpu.SemaphoreType.DMA,
            pltpu.SemaphoreType.DMA,
            pltpu.SemaphoreType.DMA,
            pltpu.SemaphoreType.DMA,
        ],
    )
    def mp_kernel(y_hbm, si_hbm, di_hbm, out_hbm,
                  didx_v, sidx4_v, rows0_v, rows1_v, acc_sh,
                  gsem0, gsem1, ssem0, ssem1, ssem2, ssem3):
        cid = lax.axis_index("c")
        sid = lax.axis_index("s")
        rows = [rows0_v, rows1_v]
        gsems = [gsem0, gsem1]
        ssems = [ssem0, ssem1, ssem2, ssem3]
        pltpu.sync_copy(di_hbm.at[pl.ds(sid * NB, NB)], didx_v)

        def g_wait(j):
            pltpu.make_async_copy(y_hbm.at[pl.ds(0, K)], rows[j],
                                  gsems[j]).wait()

        def s_wait(j):
            pltpu.make_async_copy(si_hbm.at[0], sidx4_v.at[j],
                                  ssems[j]).wait()

        for cc in range(NCHUNK // NC):
            chunk = cid * (NCHUNK // NC) + cc
            base = (chunk * NS + sid) * NB

            _zero_fill(rows0_v)
            for z in range(RPT // K):
                pltpu.sync_copy(rows0_v,
                                acc_sh.at[pl.ds(sid * RPT + z * K, K)])
            plsc.subcore_barrier()

            def s_load(b, j):
                pltpu.async_copy(si_hbm.at[base + b], sidx4_v.at[j],
                                 ssems[j])

            def gather(j_slot, j_row):
                pltpu.async_copy(y_hbm.at[sidx4_v.at[j_slot, 0]],
                                 rows[j_row], gsems[j_row])

            def scatter(b, j_row):
                pltpu.sync_copy(rows[j_row], acc_sh.at[didx_v.at[b, 0]],
                                add=True)

            for j in range(4):
                s_load(j, j)
            s_wait(0)
            gather(0, 0)
            s_wait(1)
            gather(1, 1)

            def quad(q, c):
                b = 4 * q
                for j in range(4):
                    g_wait(j % 2)
                    scatter(b + j, j % 2)
                    s_load(b + j + 4, j)
                    s_wait((j + 2) % 4)
                    gather((j + 2) % 4, j % 2)
                return c

            lax.fori_loop(0, NB // 4 - 1, quad, 0)

            bt = NB - 4
            g_wait(0)
            scatter(bt, 0)
            s_wait(2)
            gather(2, 0)
            g_wait(1)
            scatter(bt + 1, 1)
            s_wait(3)
            gather(3, 1)
            g_wait(0)
            scatter(bt + 2, 0)
            g_wait(1)
            scatter(bt + 3, 1)

            plsc.subcore_barrier()
            pltpu.sync_copy(acc_sh.at[pl.ds(sid * RPT, RPT)],
                            out_hbm.at[chunk, pl.ds(sid * RPT, RPT)])
            plsc.subcore_barrier()

    return mp_kernel(y4, srcidx, dstidx)


BN = 1000


def _norm(deg_blk):
    return lax.rsqrt(jnp.maximum(deg_blk[:, :1], 1.0))


def _layer1_body(h_ref, dout_ref, w_ref, y_ref):
    ns = _norm(dout_ref[...])
    y_ref[...] = jnp.dot(h_ref[...] * ns, w_ref[...],
                         preferred_element_type=jnp.float32)


def _layer1_call(h, deg_out, W1):
    ind = h.shape[1]
    return pl.pallas_call(
        _layer1_body,
        grid=(N // BN,),
        in_specs=[
            pl.BlockSpec((BN, ind), lambda i: (i, 0)),
            pl.BlockSpec((BN, CW), lambda i: (i, 0)),
            pl.BlockSpec((ind, HID), lambda i: (0, 0)),
        ],
        out_specs=pl.BlockSpec((BN, HID), lambda i: (i, 0)),
        out_shape=jax.ShapeDtypeStruct((N, HID), jnp.float32),
    )(h, deg_out, W1)


def _fused_body(agg_ref, din_ref, dout_ref, b_ref, w_ref, y_ref):
    nd = _norm(din_ref[...])
    ns = _norm(dout_ref[...])
    acc = None
    for c in range(NCHUNK):
        xc = jnp.maximum(agg_ref[c] * nd + b_ref[:, c * CW:(c + 1) * CW],
                         0.0) * ns
        p = jnp.dot(xc, w_ref[c * CW:(c + 1) * CW, :],
                    preferred_element_type=jnp.float32)
        acc = p if acc is None else acc + p
    y_ref[...] = acc


def _fused_call(agg4, deg_in, deg_out, b2d, W):
    return pl.pallas_call(
        _fused_body,
        grid=(N // BN,),
        in_specs=[
            pl.BlockSpec((NCHUNK, BN, CW), lambda i: (0, i, 0)),
            pl.BlockSpec((BN, CW), lambda i: (i, 0)),
            pl.BlockSpec((BN, CW), lambda i: (i, 0)),
            pl.BlockSpec((1, HID), lambda i: (0, 0)),
            pl.BlockSpec((HID, HID), lambda i: (0, 0)),
        ],
        out_specs=pl.BlockSpec((BN, HID), lambda i: (i, 0)),
        out_shape=jax.ShapeDtypeStruct((N, HID), jnp.float32),
    )(agg4, deg_in, deg_out, b2d, W)


def _final_body(agg_ref, din_ref, b_ref, wc_ref, bc_ref, out_ref, hg_ref):
    i = pl.program_id(0)
    nd = _norm(din_ref[...])
    parts = []
    for c in range(NCHUNK):
        xc = jnp.maximum(agg_ref[c] * nd + b_ref[:, c * CW:(c + 1) * CW], 0.0)
        parts.append(jnp.sum(xc, axis=0, keepdims=True))
    part = jnp.concatenate(parts, axis=1)

    @pl.when(i == 0)
    def _():
        hg_ref[...] = part

    @pl.when(i > 0)
    def _():
        hg_ref[...] = hg_ref[...] + part

    @pl.when(i == pl.num_programs(0) - 1)
    def _():
        out_ref[...] = jnp.dot(hg_ref[...], wc_ref[...],
                               preferred_element_type=jnp.float32) + bc_ref[...]


def _final_call(agg4, deg_in, b2d, Wc, bc2d):
    ncls = Wc.shape[1]
    return pl.pallas_call(
        _final_body,
        grid=(N // BN,),
        in_specs=[
            pl.BlockSpec((NCHUNK, BN, CW), lambda i: (0, i, 0)),
            pl.BlockSpec((BN, CW), lambda i: (i, 0)),
            pl.BlockSpec((1, HID), lambda i: (0, 0)),
            pl.BlockSpec((HID, ncls), lambda i: (0, 0)),
            pl.BlockSpec((1, ncls), lambda i: (0, 0)),
        ],
        out_specs=pl.BlockSpec((1, ncls), lambda i: (0, 0)),
        out_shape=jax.ShapeDtypeStruct((1, ncls), jnp.float32),
        scratch_shapes=[pltpu.VMEM((1, HID), jnp.float32)],
    )(agg4, deg_in, b2d, Wc, bc2d)


def kernel(h, edge_index, W1, b1, W2, b2, W3, b3, W4, b4, W5, b5, W6, b6,
           W7, b7, W8, b8, Wc, bc):
    edges = edge_index.astype(jnp.int32)
    src = edges[0]
    dst = edges[1]
    npad = EPAD - E
    junk = jnp.full((npad,), NPAD - 1, jnp.int32)
    edges_pad = jnp.concatenate(
        [src, junk, dst, junk]).reshape(2 * NS * NB, 1, K)
    src_pad = jnp.concatenate([src, jnp.zeros((npad,), jnp.int32)])
    dst_pad = jnp.concatenate([dst, junk])
    srcidx = (src_pad[None, :] * NCHUNK
              + jnp.arange(NCHUNK, dtype=jnp.int32)[:, None]
              ).reshape(NCHUNK * NS * NB, 1, K)
    dstidx = dst_pad.reshape(NS * NB, 1, K)

    deg = _deg_call(edges_pad)
    deg_out = deg[0]
    deg_in = deg[1]

    y = _layer1_call(h, deg_out, W1)
    bs = [b1, b2, b3, b4, b5, b6, b7, b8]
    Ws = [None, W2, W3, W4, W5, W6, W7, W8]
    for i in range(1, 8):
        agg4 = _mp_call(y.reshape(NCHUNK * N, CW), srcidx, dstidx)
        y = _fused_call(agg4, deg_in, deg_out, bs[i - 1].reshape(1, HID),
                        Ws[i])
    agg4 = _mp_call(y.reshape(NCHUNK * N, CW), srcidx, dstidx)
    return _final_call(agg4, deg_in, bs[7].reshape(1, HID), Wc,
                       bc.reshape(1, -1))


# speedup vs baseline: 2.7401x; 1.0000x over previous
import functools

import jax
import jax.numpy as jnp
from jax import lax
from jax.experimental import pallas as pl
from jax.experimental.pallas import tpu as pltpu
from jax.experimental.pallas import tpu_sc as plsc

N = 10000
NPAD = 10240
E = 160000
EPAD = 163840
HID = 512
NCHUNK = 4
CW = HID // NCHUNK
NS = 16
NC = 2
TEP = EPAD // NS
K = 128
NB = TEP // K
RPT = NPAD // NS
ZR = 128
HN = NPAD // 4
CAPB = 22
CAP = NS * K * CAPB
W2 = 256
RP2 = HN // NS


def _sc_mesh():
    return plsc.VectorSubcoreMesh(core_axis_name="c", subcore_axis_name="s")


def _zero_fill(buf):
    def fill(r, c):
        for j in range(CW // 16):
            buf[r, pl.ds(j * 16, 16)] = jnp.zeros((16,), jnp.float32)
        return c

    lax.fori_loop(0, ZR, fill, 0)


def _deg_call(edges_pad):
    @functools.partial(
        pl.kernel,
        out_type=jax.ShapeDtypeStruct((NC, NPAD, CW), jnp.float32),
        mesh=_sc_mesh(),
        scratch_types=[
            pltpu.VMEM((NB, 1, K), jnp.int32),
            pltpu.VMEM((K, CW), jnp.float32),
            pltpu.VMEM((ZR, CW), jnp.float32),
            pltpu.VMEM_SHARED((NPAD, CW), jnp.float32),
        ],
    )
    def deg_kernel(edges_hbm, deg_hbm, idx_v, ones_v, zbuf_v, acc_sh):
        cid = lax.axis_index("c")
        sid = lax.axis_index("s")

        def fill_ones(r, c):
            for j in range(CW // 16):
                ones_v[r, pl.ds(j * 16, 16)] = jnp.ones((16,), jnp.float32)
            return c

        lax.fori_loop(0, K, fill_ones, 0)
        _zero_fill(zbuf_v)

        for z in range(RPT // ZR):
            pltpu.sync_copy(zbuf_v, acc_sh.at[pl.ds(sid * RPT + z * ZR, ZR)])
        pltpu.sync_copy(edges_hbm.at[pl.ds((cid * NS + sid) * NB, NB)], idx_v)
        plsc.subcore_barrier()

        def body(b, c):
            pltpu.sync_copy(ones_v, acc_sh.at[idx_v.at[b, 0]], add=True)
            return c

        lax.fori_loop(0, NB, body, 0)
        plsc.subcore_barrier()
        pltpu.sync_copy(acc_sh.at[pl.ds(sid * RPT, RPT)],
                        deg_hbm.at[cid, pl.ds(sid * RPT, RPT)])

    return deg_kernel(edges_pad)


def _mp_call(y4, srcidx, dstidx):
    @functools.partial(
        pl.kernel,
        out_type=jax.ShapeDtypeStruct((NCHUNK, NPAD, CW), jnp.float32),
        mesh=_sc_mesh(),
        scratch_types=[
            pltpu.VMEM((NB, 1, K), jnp.int32),
            pltpu.VMEM((4, 1, K), jnp.int32),
            pltpu.VMEM((K, CW), jnp.float32),
            pltpu.VMEM((K, CW), jnp.float32),
            pltpu.VMEM_SHARED((NPAD, CW), jnp.float32),
            pltpu.SemaphoreType.DMA,
            pltpu.SemaphoreType.DMA,
            pltpu.SemaphoreType.DMA,
            pltpu.SemaphoreType.DMA,
            pltpu.SemaphoreType.DMA,
            pltpu.SemaphoreType.DMA,
        ],
    )
    def mp_kernel(y_hbm, si_hbm, di_hbm, out_hbm,
                  didx_v, sidx4_v, rows0_v, rows1_v, acc_sh,
                  gsem0, gsem1, ssem0, ssem1, ssem2, ssem3):
        cid = lax.axis_index("c")
        sid = lax.axis_index("s")
        rows = [rows0_v, rows1_v]
        gsems = [gsem0, gsem1]
        ssems = [ssem0, ssem1, ssem2, ssem3]
        pltpu.sync_copy(di_hbm.at[pl.ds(sid * NB, NB)], didx_v)

        def g_wait(j):
            pltpu.make_async_copy(y_hbm.at[pl.ds(0, K)], rows[j],
                                  gsems[j]).wait()

        def s_wait(j):
            pltpu.make_async_copy(si_hbm.at[0], sidx4_v.at[j],
                                  ssems[j]).wait()

        for cc in range(NCHUNK // NC):
            chunk = cid * (NCHUNK // NC) + cc
            base = (chunk * NS + sid) * NB

            _zero_fill(rows0_v)
            for z in range(RPT // K):
                pltpu.sync_copy(rows0_v,
                                acc_sh.at[pl.ds(sid * RPT + z * K, K)])
            plsc.subcore_barrier()

            def s_load(b, j):
                pltpu.async_copy(si_hbm.at[base + b], sidx4_v.at[j],
                                 ssems[j])

            def gather(j_slot, j_row):
                pltpu.async_copy(y_hbm.at[sidx4_v.at[j_slot, 0]],
                                 rows[j_row], gsems[j_row])

            def scatter(b, j_row):
                pltpu.sync_copy(rows[j_row], acc_sh.at[didx_v.at[b, 0]],
                                add=True)

            for j in range(4):
                s_load(j, j)
            s_wait(0)
            gather(0, 0)
            s_wait(1)
            gather(1, 1)

            def quad(q, c):
                b = 4 * q
                for j in range(4):
                    g_wait(j % 2)
                    scatter(b + j, j % 2)
                    s_load(b + j + 4, j)
                    s_wait((j + 2) % 4)
                    gather((j + 2) % 4, j % 2)
                return c

            lax.fori_loop(0, NB // 4 - 1, quad, 0)

            bt = NB - 4
            g_wait(0)
            scatter(bt, 0)
            s_wait(2)
            gather(2, 0)
            g_wait(1)
            scatter(bt + 1, 1)
            s_wait(3)
            gather(3, 1)
            g_wait(0)
            scatter(bt + 2, 0)
            g_wait(1)
            scatter(bt + 3, 1)

            plsc.subcore_barrier()
            pltpu.sync_copy(acc_sh.at[pl.ds(sid * RPT, RPT)],
                            out_hbm.at[chunk, pl.ds(sid * RPT, RPT)])
            plsc.subcore_barrier()

    return mp_kernel(y4, srcidx, dstidx)


BN = 1000


def _norm(deg_blk):
    return lax.rsqrt(jnp.maximum(deg_blk[:, :1], 1.0))


def _layer1_body(h_ref, dout_ref, w_ref, y_ref):
    ns = _norm(dout_ref[...])
    y_ref[...] = jnp.dot(h_ref[...] * ns, w_ref[...],
                         preferred_element_type=jnp.float32)


def _layer1_call(h, deg_out, W1):
    ind = h.shape[1]
    return pl.pallas_call(
        _layer1_body,
        grid=(N // BN,),
        in_specs=[
            pl.BlockSpec((BN, ind), lambda i: (i, 0)),
            pl.BlockSpec((BN, CW), lambda i: (i, 0)),
            pl.BlockSpec((ind, HID), lambda i: (0, 0)),
        ],
        out_specs=pl.BlockSpec((BN, HID), lambda i: (i, 0)),
        out_shape=jax.ShapeDtypeStruct((N, HID), jnp.float32),
    )(h, deg_out, W1)


def _fused_body(agg_ref, din_ref, dout_ref, b_ref, w_ref, y_ref):
    nd = _norm(din_ref[...])
    ns = _norm(dout_ref[...])
    acc = None
    for c in range(NCHUNK):
        xc = jnp.maximum(agg_ref[c // 2, c % 2] * nd
                         + b_ref[:, c * CW:(c + 1) * CW], 0.0) * ns
        q = jnp.dot(xc, w_ref[c * CW:(c + 1) * CW, :],
                    preferred_element_type=jnp.float32)
        acc = q if acc is None else acc + q
    y_ref[...] = acc


def _fused_call(agg2, deg_in, deg_out, b2d, W):
    return pl.pallas_call(
        _fused_body,
        grid=(N // BN,),
        in_specs=[
            pl.BlockSpec((2, 2, BN, CW), lambda i: (0, 0, i, 0)),
            pl.BlockSpec((BN, CW), lambda i: (i, 0)),
            pl.BlockSpec((BN, CW), lambda i: (i, 0)),
            pl.BlockSpec((1, HID), lambda i: (0, 0)),
            pl.BlockSpec((HID, HID), lambda i: (0, 0)),
        ],
        out_specs=pl.BlockSpec((BN, HID), lambda i: (i, 0)),
        out_shape=jax.ShapeDtypeStruct((N, HID), jnp.float32),
    )(agg2, deg_in, deg_out, b2d, W)


def _final_body(agg_ref, din_ref, b_ref, wc_ref, bc_ref, out_ref, hg_ref):
    i = pl.program_id(0)
    nd = _norm(din_ref[...])
    parts = []
    for c in range(NCHUNK):
        xc = jnp.maximum(agg_ref[c // 2, c % 2] * nd
                         + b_ref[:, c * CW:(c + 1) * CW], 0.0)
        parts.append(jnp.sum(xc, axis=0, keepdims=True))
    part = jnp.concatenate(parts, axis=1)

    @pl.when(i == 0)
    def _():
        hg_ref[...] = part

    @pl.when(i > 0)
    def _():
        hg_ref[...] = hg_ref[...] + part

    @pl.when(i == pl.num_programs(0) - 1)
    def _():
        out_ref[...] = jnp.dot(hg_ref[...], wc_ref[...],
                               preferred_element_type=jnp.float32) + bc_ref[...]


def _final_call(agg2, deg_in, b2d, Wc, bc2d):
    ncls = Wc.shape[1]
    return pl.pallas_call(
        _final_body,
        grid=(N // BN,),
        in_specs=[
            pl.BlockSpec((2, 2, BN, CW), lambda i: (0, 0, i, 0)),
            pl.BlockSpec((BN, CW), lambda i: (i, 0)),
            pl.BlockSpec((1, HID), lambda i: (0, 0)),
            pl.BlockSpec((HID, ncls), lambda i: (0, 0)),
            pl.BlockSpec((1, ncls), lambda i: (0, 0)),
        ],
        out_specs=pl.BlockSpec((1, ncls), lambda i: (0, 0)),
        out_shape=jax.ShapeDtypeStruct((1, ncls), jnp.float32),
        scratch_shapes=[pltpu.VMEM((1, HID), jnp.float32)],
    )(agg2, deg_in, b2d, Wc, bc2d)


def _edge_arrays(src, dst):
    npadE = EPAD - E
    junk = jnp.full((npadE,), NPAD - 1, jnp.int32)
    edges_pad = jnp.concatenate([src, junk, dst, junk]).reshape(
        2 * NS * NB, 1, K)
    src_pad = jnp.concatenate([src, jnp.zeros((npadE,), jnp.int32)])
    dst_pad = jnp.concatenate([dst, junk])
    srcidx4 = (src_pad[None, :] * NCHUNK
               + jnp.arange(NCHUNK, dtype=jnp.int32)[:, None]
               ).reshape(NCHUNK * NS * NB, 1, K)
    dstidx4 = dst_pad.reshape(NS * NB, 1, K)
    qid = dst // HN
    ar = jnp.arange(E, dtype=jnp.int32)
    pos = jnp.zeros((E,), jnp.int32)
    counts = []
    start = jnp.int32(0)
    for kq in range(4):
        mk = (qid == kq)
        ck = jnp.cumsum(mk.astype(jnp.int32))
        pos = jnp.where(mk, start + ck - 1, pos)
        counts.append(ck[-1])
        start = start + ck[-1]
    order = jnp.zeros((E,), jnp.int32).at[pos].set(ar)
    starts = [jnp.int32(0)]
    for kq in range(3):
        starts.append(starts[-1] + counts[kq])
    ok = (jnp.maximum(jnp.maximum(counts[0], counts[1]),
                      jnp.maximum(counts[2], counts[3])) <= CAP)
    i = jnp.arange(CAP, dtype=jnp.int32)
    s4, d4 = [], []
    for kq in range(4):
        posi = jnp.clip(starts[kq] + jnp.minimum(i, counts[kq] - 1), 0, E - 1)
        e = order[posi]
        valid = i < counts[kq]
        s4.append(jnp.where(valid, src[e], N))
        d4.append(jnp.where(valid, dst[e] - kq * HN, 0))
    s4 = jnp.stack(s4)
    d4 = jnp.stack(d4)
    srcidx2 = (2 * s4[None] + jnp.arange(2, dtype=jnp.int32)[:, None, None]
               ).reshape(2 * 4 * NS * CAPB, 1, K)
    dstidx2 = d4.reshape(4 * NS * CAPB, 1, K)
    return edges_pad, srcidx4, dstidx4, srcidx2, dstidx2, ok


def _mp(y, srcidx4, dstidx4, srcidx2, dstidx2, ok):
    del srcidx2, dstidx2, ok
    agg4 = _mp_call(y.reshape(NCHUNK * N, CW), srcidx4, dstidx4)
    return agg4.reshape(2, 2, NPAD, CW)


def kernel(h, edge_index, W1, b1, W2_, b2, W3, b3, W4, b4, W5, b5, W6, b6,
           W7, b7, W8, b8, Wc, bc):
    edges = edge_index.astype(jnp.int32)
    src = edges[0]
    dst = edges[1]
    edges_pad, srcidx4, dstidx4, srcidx2, dstidx2, ok = _edge_arrays(src, dst)

    deg = _deg_call(edges_pad)
    deg_out = deg[0]
    deg_in = deg[1]

    y = _layer1_call(h, deg_out, W1)
    bs = [b1, b2, b3, b4, b5, b6, b7, b8]
    Ws = [None, W2_, W3, W4, W5, W6, W7, W8]
    for i in range(1, 8):
        agg2 = _mp(y, srcidx4, dstidx4, srcidx2, dstidx2, ok)
        y = _fused_call(agg2, deg_in, deg_out, bs[i - 1].reshape(1, HID),
                        Ws[i])
    agg2 = _mp(y, srcidx4, dstidx4, srcidx2, dstidx2, ok)
    return _final_call(agg2, deg_in, bs[7].reshape(1, HID), Wc,
                       bc.reshape(1, -1))
